# baseline (speedup 1.0000x reference)
import functools

import jax
import jax.numpy as jnp
from jax import lax
from jax.experimental import pallas as pl
from jax.experimental.pallas import tpu as pltpu
from jax.experimental.pallas import tpu_sc as plsc

EMB = 64
SCALE = 8.0

_info = plsc.get_sparse_core_info()
_NC = _info.num_cores
_NS = _info.num_subcores
_NW = _NC * _NS

_B = 256
_NG = _B // 128


@functools.lru_cache(maxsize=None)
def _make_lookup(n_cols: int, n_rows: int, emb: int):
    n_blk = n_rows // _B
    tasks = n_cols * n_blk
    per_w = tasks // _NW
    mesh = plsc.VectorSubcoreMesh(core_axis_name="c", subcore_axis_name="s")

    @functools.partial(
        pl.kernel,
        out_type=jax.ShapeDtypeStruct((n_cols, emb, n_rows), jnp.float32),
        mesh=mesh,
        compiler_params=pltpu.CompilerParams(needs_layout_passes=False),
        scratch_types=(
            [pltpu.VMEM((_B,), jnp.int32) for _ in range(2)]
            + [pltpu.VMEM((_NG, 128), jnp.int32) for _ in range(2)]
            + [pltpu.VMEM((_B, 128), jnp.float32) for _ in range(2)]
            + [pltpu.VMEM((emb, _B), jnp.float32) for _ in range(2)]
            + [pltpu.SemaphoreType.DMA for _ in range(6)]
        ),
    )
    def lookup(tok_hbm, pair_hbm, out_hbm, *rest):
        idxb = rest[0:2]
        pidxb = rest[2:4]
        inb = rest[4:6]
        outb = rest[6:8]
        isem = rest[8:10]
        gsem = rest[10:12]
        osem = rest[12:14]
        wid = lax.axis_index("c") * _NS + lax.axis_index("s")
        lanes = lax.broadcasted_iota(jnp.int32, (16,), 0)

        def coords(t):
            tg = wid * per_w + t
            j = tg // n_blk
            i0 = (tg % n_blk) * _B
            return j, i0

        def idx_src(t):
            j, i0 = coords(t)
            return tok_hbm.at[j, pl.ds(i0, _B)]

        def start_idx(t, b):
            pltpu.async_copy(idx_src(t), idxb[b], isem[b])

        def compute_pidx(b):
            for m in range(_B // 16):
                v = idxb[b][pl.ds(m * 16, 16)]
                pidxb[b][m // 8, pl.ds((m % 8) * 16, 16)] = (
                    lax.shift_right_logical(v, 1)
                )

        def start_gather(b):
            for k in range(_NG):
                pltpu.async_copy(
                    pair_hbm.at[pidxb[b].at[k]],
                    inb[b].at[pl.ds(k * 128, 128)],
                    gsem[b],
                )

        def wait_gather(b):
            for k in range(_NG):
                pltpu.make_async_copy(
                    pair_hbm.at[pidxb[b].at[k]],
                    inb[b].at[pl.ds(k * 128, 128)],
                    gsem[b],
                ).wait()

        def transpose_scale(b):
            @pl.loop(0, _B // 16)
            def _g(g):
                tok = idxb[b][pl.ds(g * 16, 16)]
                col0 = lax.shift_left(
                    lax.bitwise_and(tok, jnp.int32(1)), jnp.int32(6)
                )
                rowv = g * 16 + lanes

                @pl.loop(0, emb, init_carry=col0, unroll=8)
                def _c(c, colv):
                    vals = plsc.load_gather(inb[b], [rowv, colv])
                    outb[b][c, pl.ds(g * 16, 16)] = vals * SCALE
                    return colv + 1

        def out_dst(t):
            j, i0 = coords(t)
            return out_hbm.at[j, :, pl.ds(i0, _B)]

        def start_out(t, b):
            pltpu.async_copy(outb[b], out_dst(t), osem[b])

        def wait_out(t, b):
            pltpu.make_async_copy(outb[b], out_dst(t), osem[b]).wait()

        start_idx(0, 0)
        start_idx(1, 1)
        pltpu.make_async_copy(idx_src(0), idxb[0], isem[0]).wait()
        compute_pidx(0)
        start_gather(0)

        @pl.loop(0, per_w // 2)
        def _outer(tt):
            for k in range(2):
                t = tt * 2 + k
                b = k

                @pl.when(t >= 2)
                def _():
                    wait_out(t - 2, b)

                wait_gather(b)
                transpose_scale(b)

                @pl.when(t + 1 < per_w)
                def _():
                    pltpu.make_async_copy(
                        idx_src(t + 1), idxb[1 - b], isem[1 - b]
                    ).wait()
                    compute_pidx(1 - b)
                    start_gather(1 - b)

                @pl.when(t + 2 < per_w)
                def _():
                    start_idx(t + 2, b)

                start_out(t, b)

        wait_out(per_w - 2, 0)
        wait_out(per_w - 1, 1)

    return lookup


def kernel(tokens, embedding_weight):
    n_rows, n_cols = tokens.shape
    tok_t = tokens.astype(jnp.int32).T
    pair_view = embedding_weight.reshape(-1, 2 * EMB)
    out_l = _make_lookup(n_cols, n_rows, EMB)(tok_t, pair_view)
    return jnp.transpose(out_l, (2, 0, 1))
